# baseline (speedup 1.0000x reference)
import functools

import jax
import jax.numpy as jnp
from jax import lax
from jax.experimental import pallas as pl
from jax.experimental.pallas import tpu as pltpu
from jax.experimental.pallas import tpu_sc as plsc

_D = 128
_WINDOW = 128
_TBL = 128
_R_TC = 2048
_N_SC = 614400


def _prep_body(d_ids_ref, s_ids_ref, dt_ref, st_ref, t3_ref, cidx_ref):
    cidx_ref[...] = d_ids_ref[...] * 50 + s_ids_ref[...]
    dt = dt_ref[...]
    st = st_ref[...]
    t3_ref[...] = dt[:, None, :] + st[None, :, :]


def _prep(d_ids2, s_ids2, depth_table, subtree_table):
    nd, d_model = depth_table.shape
    ns = subtree_table.shape[0]
    t3, cidx2 = pl.pallas_call(
        _prep_body,
        out_shape=[
            jax.ShapeDtypeStruct((nd, ns, d_model), jnp.float32),
            jax.ShapeDtypeStruct(d_ids2.shape, jnp.int32),
        ],
    )(d_ids2, s_ids2, depth_table, subtree_table)
    return t3.reshape(nd * ns, d_model), cidx2


def _sc_gather(table, cidx2, n):
    mesh = plsc.VectorSubcoreMesh(core_axis_name="c", subcore_axis_name="s")

    @functools.partial(
        pl.kernel,
        out_type=jax.ShapeDtypeStruct((n, _D), jnp.float32),
        mesh=mesh,
        scratch_types=[pltpu.VMEM_SHARED((1000, _D), jnp.float32)],
    )
    def k(tbl_hbm, idx_hbm, out_hbm, tbl_sh):
        @pl.when(lax.axis_index("s") == 0)
        def _():
            pltpu.sync_copy(tbl_hbm, tbl_sh)

        plsc.subcore_barrier()

        def body(i_vmem, o_vmem):
            pltpu.sync_copy(tbl_sh.at[i_vmem.at[0]], o_vmem)

        pltpu.emit_pipeline(
            body,
            grid=(n // _WINDOW,),
            in_specs=[pl.BlockSpec((1, _WINDOW), lambda i: (i, 0))],
            out_specs=[pl.BlockSpec((_WINDOW, _D), lambda i: (i, 0))],
            core_axis_name=("c", "s"),
            dimension_semantics=(pltpu.PARALLEL,),
        )(idx_hbm, out_hbm)

    return k(table, cidx2)


def _tc_embed_block(d_ref, s_ref, t_ref, o_ref):
    r = d_ref.shape[0]
    d = d_ref[...]
    s = s_ref[...]
    col = lax.broadcasted_iota(jnp.int32, (r, _TBL), 1)
    onehot = (col == d).astype(jnp.bfloat16) + (col == (s + 20)).astype(jnp.bfloat16)
    t = t_ref[...]
    t_hi = t.astype(jnp.bfloat16)
    t_lo = (t - t_hi.astype(jnp.float32)).astype(jnp.bfloat16)
    acc = jnp.dot(onehot, t_hi, preferred_element_type=jnp.float32)
    acc = acc + jnp.dot(onehot, t_lo, preferred_element_type=jnp.float32)
    o_ref[...] = acc


def _tc_embed(d_col, s_col, tcat, n, d_model):
    return pl.pallas_call(
        _tc_embed_block,
        grid=(n // _R_TC,),
        in_specs=[
            pl.BlockSpec((_R_TC, 1), lambda i: (i, 0)),
            pl.BlockSpec((_R_TC, 1), lambda i: (i, 0)),
            pl.BlockSpec((_TBL, d_model), lambda i: (0, 0)),
        ],
        out_specs=pl.BlockSpec((_R_TC, d_model), lambda i: (i, 0)),
        out_shape=jax.ShapeDtypeStruct((n, d_model), jnp.float32),
    )(d_col, s_col, tcat)


def kernel(depth_ids, subtree_ids, depth_table, subtree_table):
    b, sq = depth_ids.shape
    nd, d_model = depth_table.shape
    ns = subtree_table.shape[0]
    n = b * sq
    n_tc = n - _N_SC

    d_flat = depth_ids.reshape(n).astype(jnp.int32)
    s_flat = subtree_ids.reshape(n).astype(jnp.int32)

    table, cidx2 = _prep(
        d_flat[:_N_SC].reshape(_N_SC // _WINDOW, _WINDOW),
        s_flat[:_N_SC].reshape(_N_SC // _WINDOW, _WINDOW),
        depth_table,
        subtree_table,
    )
    sc_out = _sc_gather(table, cidx2, _N_SC)

    tcat = jnp.zeros((_TBL, d_model), jnp.float32)
    tcat = tcat.at[:nd].set(depth_table)
    tcat = tcat.at[20 : 20 + ns].set(subtree_table)
    tc_out = _tc_embed(
        d_flat[_N_SC:].reshape(n_tc, 1),
        s_flat[_N_SC:].reshape(n_tc, 1),
        tcat,
        n_tc,
        d_model,
    )

    out = jnp.concatenate([sc_out, tc_out], axis=0)
    return out.reshape(b, sq, d_model)


# speedup vs baseline: 2.7670x; 2.7670x over previous
import functools

import jax
import jax.numpy as jnp
from jax import lax
from jax.experimental import pallas as pl
from jax.experimental.pallas import tpu as pltpu
from jax.experimental.pallas import tpu_sc as plsc

_D = 128
_WINDOW = 128
_LANES = 16
_ND = 20
_NS = 50


def _sc_embed(depth_table, subtree_table, d_ids2, s_ids2, n):
    mesh = plsc.VectorSubcoreMesh(core_axis_name="c", subcore_axis_name="s")

    @functools.partial(
        pl.kernel,
        out_type=jax.ShapeDtypeStruct((n, _D), jnp.float32),
        mesh=mesh,
        scratch_types=[
            pltpu.VMEM_SHARED((_ND * _NS, _D), jnp.float32),
            pltpu.VMEM((_ND, _D), jnp.float32),
            pltpu.VMEM((_NS, _D), jnp.float32),
            pltpu.VMEM((_NS, _D), jnp.float32),
            pltpu.VMEM((_WINDOW,), jnp.int32),
        ],
    )
    def k(dt_hbm, st_hbm, d_hbm, s_hbm, out_hbm, t3_sh, dt_v, st_v, tb_v, idx_v):
        sid = lax.axis_index("s")
        pltpu.sync_copy(dt_hbm, dt_v)
        pltpu.sync_copy(st_hbm, st_v)

        def build(d):
            @pl.loop(0, _NS)
            def _(srow):
                @pl.loop(0, _D, step=_LANES)
                def _(c):
                    tb_v[srow, pl.ds(c, _LANES)] = (
                        dt_v[d, pl.ds(c, _LANES)] + st_v[srow, pl.ds(c, _LANES)]
                    )

            pltpu.sync_copy(tb_v, t3_sh.at[pl.ds(d * _NS, _NS)])

        build(sid)

        @pl.when(sid < _ND - 16)
        def _():
            build(sid + 16)

        plsc.subcore_barrier()

        def body(d_v, s_v, o_vmem):
            @pl.loop(0, _WINDOW, step=_LANES)
            def _(j):
                dd = d_v[0, pl.ds(j, _LANES)]
                ss = s_v[0, pl.ds(j, _LANES)]
                idx_v[pl.ds(j, _LANES)] = dd * _NS + ss

            pltpu.sync_copy(t3_sh.at[idx_v], o_vmem)

        pltpu.emit_pipeline(
            body,
            grid=(n // _WINDOW,),
            in_specs=[
                pl.BlockSpec((1, _WINDOW), lambda i: (i, 0)),
                pl.BlockSpec((1, _WINDOW), lambda i: (i, 0)),
            ],
            out_specs=[pl.BlockSpec((_WINDOW, _D), lambda i: (i, 0))],
            core_axis_name=("c", "s"),
            dimension_semantics=(pltpu.PARALLEL,),
        )(d_hbm, s_hbm, out_hbm)

    return k(depth_table, subtree_table, d_ids2, s_ids2)


def kernel(depth_ids, subtree_ids, depth_table, subtree_table):
    b, sq = depth_ids.shape
    d_model = depth_table.shape[1]
    n = b * sq

    d_ids2 = depth_ids.reshape(n // _WINDOW, _WINDOW).astype(jnp.int32)
    s_ids2 = subtree_ids.reshape(n // _WINDOW, _WINDOW).astype(jnp.int32)

    out = _sc_embed(depth_table, subtree_table, d_ids2, s_ids2, n)
    return out.reshape(b, sq, d_model)


# speedup vs baseline: 2.8275x; 1.0218x over previous
import functools

import jax
import jax.numpy as jnp
from jax import lax
from jax.experimental import pallas as pl
from jax.experimental.pallas import tpu as pltpu
from jax.experimental.pallas import tpu_sc as plsc

_D = 128
_WINDOW = 128
_NTBL = 1024


def _prep_body(d_ids_ref, s_ids_ref, dt_ref, st_ref, t3_ref, cidx_ref):
    cidx_ref[...] = d_ids_ref[...] * 50 + s_ids_ref[...]

    @pl.when(pl.program_id(0) == 0)
    def _():
        dt = dt_ref[...]
        st = st_ref[...]
        t3_ref[...] = dt[:, None, :] + st[None, :, :]


def _sc_gather(table, cidx, n):
    mesh = plsc.VectorSubcoreMesh(core_axis_name="c", subcore_axis_name="s")

    @functools.partial(
        pl.kernel,
        out_type=jax.ShapeDtypeStruct((n, _D), jnp.float32),
        mesh=mesh,
        scratch_types=[pltpu.VMEM_SHARED((1000, _D), jnp.float32)],
    )
    def k(tbl_hbm, idx_hbm, out_hbm, tbl_sh):
        @pl.when(lax.axis_index("s") == 0)
        def _():
            pltpu.sync_copy(tbl_hbm, tbl_sh)

        plsc.subcore_barrier()

        def body(i_vmem, o_vmem):
            pltpu.sync_copy(tbl_sh.at[i_vmem.at[0]], o_vmem)

        pltpu.emit_pipeline(
            body,
            grid=(n // _WINDOW,),
            in_specs=[pl.BlockSpec((1, _WINDOW), lambda i: (0, i))],
            out_specs=[pl.BlockSpec((_WINDOW, _D), lambda i: (i, 0))],
            core_axis_name=("c", "s"),
            dimension_semantics=(pltpu.PARALLEL,),
        )(idx_hbm, out_hbm)

    return k(table, cidx)


def kernel(depth_ids, subtree_ids, depth_table, subtree_table):
    b, sq = depth_ids.shape
    nd, d_model = depth_table.shape
    ns = subtree_table.shape[0]
    n = b * sq

    d_ids2 = depth_ids.reshape(n // 128, 128).astype(jnp.int32)
    s_ids2 = subtree_ids.reshape(n // 128, 128).astype(jnp.int32)

    nb = n // 128 // 8
    t3, cidx2 = pl.pallas_call(
        _prep_body,
        grid=(8,),
        in_specs=[
            pl.BlockSpec((nb, 128), lambda i: (i, 0)),
            pl.BlockSpec((nb, 128), lambda i: (i, 0)),
            pl.BlockSpec((nd, d_model), lambda i: (0, 0)),
            pl.BlockSpec((ns, d_model), lambda i: (0, 0)),
        ],
        out_specs=[
            pl.BlockSpec((nd, ns, d_model), lambda i: (0, 0, 0)),
            pl.BlockSpec((nb, 128), lambda i: (i, 0)),
        ],
        out_shape=[
            jax.ShapeDtypeStruct((nd, ns, d_model), jnp.float32),
            jax.ShapeDtypeStruct((n // 128, 128), jnp.int32),
        ],
    )(d_ids2, s_ids2, depth_table, subtree_table)

    table = t3.reshape(nd * ns, d_model)
    cidx = cidx2.reshape(1, n)

    out = _sc_gather(table, cidx, n)
    return out.reshape(b, sq, d_model)


# speedup vs baseline: 2.8603x; 1.0116x over previous
import functools

import jax
import jax.numpy as jnp
from jax import lax
from jax.experimental import pallas as pl
from jax.experimental.pallas import tpu as pltpu
from jax.experimental.pallas import tpu_sc as plsc

_D = 128
_WINDOW = 128


def _prep_body(d_ids_ref, s_ids_ref, dt_ref, st_ref, t3_ref, cidx_ref):
    cidx_ref[...] = d_ids_ref[...] * 50 + s_ids_ref[...]
    dt = dt_ref[...]
    st = st_ref[...]
    t3_ref[...] = dt[:, None, :] + st[None, :, :]


def _sc_gather(table, cidx, n):
    mesh = plsc.VectorSubcoreMesh(core_axis_name="c", subcore_axis_name="s")

    @functools.partial(
        pl.kernel,
        out_type=jax.ShapeDtypeStruct((n, _D), jnp.float32),
        mesh=mesh,
        scratch_types=[pltpu.VMEM_SHARED((1000, _D), jnp.float32)],
    )
    def k(tbl_hbm, idx_hbm, out_hbm, tbl_sh):
        @pl.when(lax.axis_index("s") == 0)
        def _():
            pltpu.sync_copy(tbl_hbm, tbl_sh)

        plsc.subcore_barrier()

        def body(i_vmem, o_vmem):
            pltpu.sync_copy(tbl_sh.at[i_vmem.at[0]], o_vmem)

        pltpu.emit_pipeline(
            body,
            grid=(n // _WINDOW,),
            in_specs=[pl.BlockSpec((1, _WINDOW), lambda i: (0, i))],
            out_specs=[pl.BlockSpec((_WINDOW, _D), lambda i: (i, 0))],
            core_axis_name=("c", "s"),
            dimension_semantics=(pltpu.PARALLEL,),
        )(idx_hbm, out_hbm)

    return k(table, cidx)


def kernel(depth_ids, subtree_ids, depth_table, subtree_table):
    b, sq = depth_ids.shape
    nd, d_model = depth_table.shape
    ns = subtree_table.shape[0]
    n = b * sq

    d_ids2 = depth_ids.reshape(n // 128, 128).astype(jnp.int32)
    s_ids2 = subtree_ids.reshape(n // 128, 128).astype(jnp.int32)

    t3, cidx2 = pl.pallas_call(
        _prep_body,
        out_shape=[
            jax.ShapeDtypeStruct((nd, ns, d_model), jnp.float32),
            jax.ShapeDtypeStruct((n // 128, 128), jnp.int32),
        ],
    )(d_ids2, s_ids2, depth_table, subtree_table)

    table = t3.reshape(nd * ns, d_model)
    cidx = cidx2.reshape(1, n)

    out = _sc_gather(table, cidx, n)
    return out.reshape(b, sq, d_model)
